# baseline (speedup 1.0000x reference)
import functools

import jax
import jax.numpy as jnp
from jax import lax
from jax.experimental import pallas as pl
from jax.experimental.pallas import tpu as pltpu
from jax.experimental.pallas import tpu_sc as plsc
import numpy as np

F = 128
N_NODES = 10000
N_EDGES = 320000
AVG_DEG_LOG = float(np.log(33.0))

NB = 96
NPB = 112
NPAD = NB * NPB
NROUND = NB // 32
LCAP = 4480
G = 64
LLEN = LCAP + 3 * G + 16
CH = 2000
NCH = N_EDGES // CH
EBLK = 4000
NBLK4 = 768


def _prep_body(x_ref, wpre_ref, wedge_ref, bedge_ref, bpre_ref, wpost_ref,
               bpost_ref, wlin_ref, blin_ref,
               a_ref, b_ref, wec_ref, c0_ref, p_ref, bout_ref):
    wp1 = wpre_ref[0:F, :]
    wp2 = wpre_ref[F:2 * F, :]
    wp3 = wpre_ref[2 * F:3 * F, :]
    x = x_ref[...]
    a_ref[...] = jnp.dot(x, wp1, preferred_element_type=jnp.float32)
    b_ref[...] = jnp.dot(x, wp2, preferred_element_type=jnp.float32)
    wec_ref[...] = jnp.dot(wedge_ref[...], wp3, preferred_element_type=jnp.float32)
    c0_ref[...] = (jnp.dot(bedge_ref[...], wp3, preferred_element_type=jnp.float32)
                   + bpre_ref[...])
    wlin = wlin_ref[...]
    p_ref[...] = jnp.dot(wpost_ref[...], wlin, preferred_element_type=jnp.float32)
    bout_ref[...] = (jnp.dot(bpost_ref[...], wlin, preferred_element_type=jnp.float32)
                     + blin_ref[...])


def _cmat_body(ea_ref, wec_ref, c0_ref, c_ref):
    c_ref[...] = (jnp.dot(ea_ref[...], wec_ref[...],
                          preferred_element_type=jnp.float32) + c0_ref[...])


def _sc_body(src_hbm, dst_hbm, b_hbm, c_hbm,
             cnt_hbm, s_hbm, s2_hbm, mx_hbm, mn_hbm,
             dbuf0, dbuf1, sbuf0, sbuf1, ldst, lsrc, lid,
             brow0, brow1, crow0, crow1,
             acc_s, acc_s2, acc_mx, acc_mn, acc_c,
             sem_b0, sem_c0, sem_b1, sem_c1):
    cid = lax.axis_index("c")
    sid = lax.axis_index("s")
    wid = sid * 2 + cid

    zf = jnp.zeros((16,), jnp.float32)
    zi = jnp.zeros((16,), jnp.int32)
    neg = jnp.full((16,), -3.0e38, jnp.float32)
    big = jnp.full((16,), 3.0e38, jnp.float32)
    iota16 = lax.iota(jnp.int32, 16)
    onesf = jnp.ones((16,), jnp.float32)
    lane0 = iota16 == 0

    def init_lists(i, _):
        o = i * 16
        ldst[pl.ds(o, 16)] = zi
        lsrc[pl.ds(o, 16)] = zi
        lid[pl.ds(o, 16)] = zi
        return 0
    lax.fori_loop(0, LLEN // 16, init_lists, 0)

    for r in range(NROUND):
        kb = wid + 32 * r
        lo = kb * NPB
        lo128 = lo * F

        def init_acc(i, _):
            o = i * 16
            acc_s[pl.ds(o, 16)] = zf
            acc_s2[pl.ds(o, 16)] = zf
            acc_mx[pl.ds(o, 16)] = neg
            acc_mn[pl.ds(o, 16)] = big
            return 0
        lax.fori_loop(0, (NPB + 1) * F // 16, init_acc, 0)

        def init_cnt(i, _):
            acc_c[pl.ds(i * 16, 16)] = zf
            return 0
        lax.fori_loop(0, (NPB + 16) // 16, init_cnt, 0)

        def scan_inner(db, sb, base, pv0):
            def scan_v(j, pv):
                d = db[pl.ds(j * 16, 16)]
                s = sb[pl.ds(j * 16, 16)]
                m = (d >= lo) & (d < lo + NPB)
                csum = plsc.cumsum(m.astype(jnp.int32))
                posn = pv + csum - 1
                m2 = m & (posn < LCAP)
                plsc.store_scatter(ldst, [posn], d, mask=m2)
                plsc.store_scatter(lsrc, [posn], s, mask=m2)
                ids = base + j * 16 + iota16
                plsc.store_scatter(lid, [posn], ids, mask=m2)
                return pv + plsc.all_reduce_population_count(m)
            return lax.fori_loop(0, CH // 16, scan_v, pv0)

        def issue_scan(ch, db, sb, semd, sems):
            base = jnp.minimum(ch, NCH - 1) * CH
            pltpu.async_copy(dst_hbm.at[pl.ds(base, CH)], db, semd)
            pltpu.async_copy(src_hbm.at[pl.ds(base, CH)], sb, sems)

        def wait_scan(db, sb, semd, sems):
            pltpu.make_async_copy(dst_hbm.at[pl.ds(0, CH)], db, semd).wait()
            pltpu.make_async_copy(src_hbm.at[pl.ds(0, CH)], sb, sems).wait()

        issue_scan(0, dbuf0, sbuf0, sem_b0, sem_c0)

        def scan_pair(i, pv):
            c0 = 2 * i
            issue_scan(c0 + 1, dbuf1, sbuf1, sem_b1, sem_c1)
            wait_scan(dbuf0, sbuf0, sem_b0, sem_c0)
            pv = scan_inner(dbuf0, sbuf0, c0 * CH, pv)
            issue_scan(c0 + 2, dbuf0, sbuf0, sem_b0, sem_c0)
            wait_scan(dbuf1, sbuf1, sem_b1, sem_c1)
            pv = scan_inner(dbuf1, sbuf1, (c0 + 1) * CH, pv)
            return pv

        ptrv = lax.fori_loop(0, NCH // 2, scan_pair, zi)
        wait_scan(dbuf0, sbuf0, sem_b0, sem_c0)
        n = jnp.minimum(ptrv[0], LCAP)

        junkv = jnp.full((16,), lo + NPB, jnp.int32)

        def padk(k, _):
            posn = n + k * 16 + iota16
            plsc.store_scatter(ldst, [posn], junkv, mask=posn < LLEN)
            return 0
        lax.fori_loop(0, 2 * G // 16, padk, 0)

        nch = (n + G - 1) // G
        nhalf = (nch + 1) // 2

        def issue_gather(g, br, cr, semb, semc):
            gb = g * G
            pltpu.async_copy(b_hbm.at[lsrc.at[pl.ds(gb, G)]], br, semb)
            pltpu.async_copy(c_hbm.at[lid.at[pl.ds(gb, G)]], cr, semc)

        def wait_gather(br, cr, semb, semc):
            pltpu.make_async_copy(b_hbm.at[lsrc.at[pl.ds(0, G)]], br, semb).wait()
            pltpu.make_async_copy(c_hbm.at[lid.at[pl.ds(0, G)]], cr, semc).wait()

        def process(br, cr, gb):
            def per_group(gi, _):
                gbase = gb + gi * 16
                dvec = ldst[pl.ds(gbase, 16)]
                for es in range(16):
                    rel = dvec[es] - lo
                    plsc.addupdate_scatter(
                        acc_c, [jnp.full((16,), rel, jnp.int32)], onesf,
                        mask=lane0)
                    fbase = rel * F
                    erow = gi * 16 + es
                    for f in range(F // 16):
                        t = (br[erow, pl.ds(f * 16, 16)]
                             + cr[erow, pl.ds(f * 16, 16)])
                        off = pl.ds(fbase + f * 16, 16)
                        plsc.addupdate(acc_s.at[off], t)
                        plsc.addupdate(acc_s2.at[off], t * t)
                        acc_mx[off] = jnp.maximum(acc_mx[off], t)
                        acc_mn[off] = jnp.minimum(acc_mn[off], t)
                return 0
            lax.fori_loop(0, G // 16, per_group, 0)

        issue_gather(0, brow0, crow0, sem_b0, sem_c0)

        def acc_pair(i, _):
            g0 = 2 * i
            issue_gather(g0 + 1, brow1, crow1, sem_b1, sem_c1)
            wait_gather(brow0, crow0, sem_b0, sem_c0)
            process(brow0, crow0, g0 * G)
            issue_gather(g0 + 2, brow0, crow0, sem_b0, sem_c0)
            wait_gather(brow1, crow1, sem_b1, sem_c1)
            process(brow1, crow1, (g0 + 1) * G)
            return 0

        lax.fori_loop(0, nhalf, acc_pair, 0)
        wait_gather(brow0, crow0, sem_b0, sem_c0)

        pltpu.sync_copy(acc_s.at[pl.ds(0, NPB * F)], s_hbm.at[pl.ds(lo128, NPB * F)])
        pltpu.sync_copy(acc_s2.at[pl.ds(0, NPB * F)], s2_hbm.at[pl.ds(lo128, NPB * F)])
        pltpu.sync_copy(acc_mx.at[pl.ds(0, NPB * F)], mx_hbm.at[pl.ds(lo128, NPB * F)])
        pltpu.sync_copy(acc_mn.at[pl.ds(0, NPB * F)], mn_hbm.at[pl.ds(lo128, NPB * F)])
        pltpu.sync_copy(acc_c.at[pl.ds(0, NPB)], cnt_hbm.at[pl.ds(lo, NPB)])


def _post_body(x_ref, a_ref, cnt_ref, s_ref, s2_ref, mx_ref, mn_ref,
               p_ref, bout_ref, o_ref):
    cnt = cnt_ref[...]
    has = cnt > 0.0
    cntc = jnp.maximum(cnt, 1.0)
    inv = 1.0 / cntc
    a = a_ref[...]
    s = s_ref[...]
    mt = s * inv
    mean = jnp.where(has, a + mt, 0.0)
    mx = jnp.where(has, a + mx_ref[...], 0.0)
    mn = jnp.where(has, a + mn_ref[...], 0.0)
    var = s2_ref[...] * inv - mt * mt
    std = jnp.sqrt(jnp.maximum(var, 0.0) + 1e-5)
    agg = jnp.concatenate([mean, mx, mn, std], axis=1)
    degl = jnp.log(cntc + 1.0)
    s_amp = degl * (1.0 / AVG_DEG_LOG)
    s_att = AVG_DEG_LOG / degl
    p = p_ref[...]
    out = jnp.dot(x_ref[...], p[0:F, :], preferred_element_type=jnp.float32)
    out += jnp.dot(agg, p[F:5 * F, :], preferred_element_type=jnp.float32)
    out += s_amp * jnp.dot(agg, p[5 * F:9 * F, :], preferred_element_type=jnp.float32)
    out += s_att * jnp.dot(agg, p[9 * F:13 * F, :], preferred_element_type=jnp.float32)
    o_ref[...] = out + bout_ref[...]


def kernel(x, edge_index, edge_attr, W_edge, b_edge, W_pre, b_pre,
           W_post, b_post, W_lin, b_lin):
    x_pad = jnp.pad(x, ((0, NPAD - N_NODES), (0, 0)))

    prep = pl.pallas_call(
        _prep_body,
        out_shape=[
            jax.ShapeDtypeStruct((NPAD, F), jnp.float32),
            jax.ShapeDtypeStruct((NPAD, F), jnp.float32),
            jax.ShapeDtypeStruct((10, F), jnp.float32),
            jax.ShapeDtypeStruct((1, F), jnp.float32),
            jax.ShapeDtypeStruct((13 * F, F), jnp.float32),
            jax.ShapeDtypeStruct((1, F), jnp.float32),
        ],
    )
    a_mat, b_mat, w_ec, c0, p_mat, b_out = prep(
        x_pad, W_pre, W_edge, b_edge.reshape(1, F), b_pre.reshape(1, F),
        W_post, b_post.reshape(1, F), W_lin, b_lin.reshape(1, F))

    cmat = pl.pallas_call(
        _cmat_body,
        grid=(N_EDGES // EBLK,),
        in_specs=[
            pl.BlockSpec((EBLK, 10), lambda i: (i, 0)),
            pl.BlockSpec((10, F), lambda i: (0, 0)),
            pl.BlockSpec((1, F), lambda i: (0, 0)),
        ],
        out_specs=pl.BlockSpec((EBLK, F), lambda i: (i, 0)),
        out_shape=jax.ShapeDtypeStruct((N_EDGES, F), jnp.float32),
    )
    c_mat = cmat(edge_attr, w_ec, c0)

    mesh = plsc.VectorSubcoreMesh(core_axis_name="c", subcore_axis_name="s")
    sc = pl.kernel(
        _sc_body,
        out_type=[
            jax.ShapeDtypeStruct((NPAD,), jnp.float32),
            jax.ShapeDtypeStruct((NPAD * F,), jnp.float32),
            jax.ShapeDtypeStruct((NPAD * F,), jnp.float32),
            jax.ShapeDtypeStruct((NPAD * F,), jnp.float32),
            jax.ShapeDtypeStruct((NPAD * F,), jnp.float32),
        ],
        mesh=mesh,
        scratch_types=[
            pltpu.VMEM((CH,), jnp.int32),
            pltpu.VMEM((CH,), jnp.int32),
            pltpu.VMEM((CH,), jnp.int32),
            pltpu.VMEM((CH,), jnp.int32),
            pltpu.VMEM((LLEN,), jnp.int32),
            pltpu.VMEM((LLEN,), jnp.int32),
            pltpu.VMEM((LLEN,), jnp.int32),
            pltpu.VMEM((G, F), jnp.float32),
            pltpu.VMEM((G, F), jnp.float32),
            pltpu.VMEM((G, F), jnp.float32),
            pltpu.VMEM((G, F), jnp.float32),
            pltpu.VMEM(((NPB + 1) * F,), jnp.float32),
            pltpu.VMEM(((NPB + 1) * F,), jnp.float32),
            pltpu.VMEM(((NPB + 1) * F,), jnp.float32),
            pltpu.VMEM(((NPB + 1) * F,), jnp.float32),
            pltpu.VMEM((NPB + 16,), jnp.float32),
            pltpu.SemaphoreType.DMA,
            pltpu.SemaphoreType.DMA,
            pltpu.SemaphoreType.DMA,
            pltpu.SemaphoreType.DMA,
        ],
        compiler_params=pltpu.CompilerParams(needs_layout_passes=False),
    )
    cnt, s_flat, s2_flat, mx_flat, mn_flat = sc(
        edge_index[0], edge_index[1], b_mat, c_mat)

    post = pl.pallas_call(
        _post_body,
        grid=(NPAD // NBLK4,),
        in_specs=[
            pl.BlockSpec((NBLK4, F), lambda i: (i, 0)),
            pl.BlockSpec((NBLK4, F), lambda i: (i, 0)),
            pl.BlockSpec((NBLK4, 1), lambda i: (i, 0)),
            pl.BlockSpec((NBLK4, F), lambda i: (i, 0)),
            pl.BlockSpec((NBLK4, F), lambda i: (i, 0)),
            pl.BlockSpec((NBLK4, F), lambda i: (i, 0)),
            pl.BlockSpec((NBLK4, F), lambda i: (i, 0)),
            pl.BlockSpec((13 * F, F), lambda i: (0, 0)),
            pl.BlockSpec((1, F), lambda i: (0, 0)),
        ],
        out_specs=pl.BlockSpec((NBLK4, F), lambda i: (i, 0)),
        out_shape=jax.ShapeDtypeStruct((NPAD, F), jnp.float32),
    )
    out = post(x_pad, a_mat, cnt.reshape(NPAD, 1),
               s_flat.reshape(NPAD, F), s2_flat.reshape(NPAD, F),
               mx_flat.reshape(NPAD, F), mn_flat.reshape(NPAD, F),
               p_mat, b_out)
    return out[:N_NODES]


# speedup vs baseline: 1.0263x; 1.0263x over previous
import functools

import jax
import jax.numpy as jnp
from jax import lax
from jax.experimental import pallas as pl
from jax.experimental.pallas import tpu as pltpu
from jax.experimental.pallas import tpu_sc as plsc
import numpy as np

F = 128
N_NODES = 10000
N_EDGES = 320000
AVG_DEG_LOG = float(np.log(33.0))

NB = 96
NPB = 112
NPAD = NB * NPB
NROUND = NB // 32
LCAP = 6144
G = 64
LLEN = LCAP + 3 * G + 16
CH = 3200
NCH = N_EDGES // CH
EBLK = 4000
NBLK4 = 768


def _prep_body(x_ref, wpre_ref, wedge_ref, bedge_ref, bpre_ref, wpost_ref,
               bpost_ref, wlin_ref, blin_ref,
               a_ref, b_ref, wec_ref, c0_ref, p_ref, bout_ref):
    wp1 = wpre_ref[0:F, :]
    wp2 = wpre_ref[F:2 * F, :]
    wp3 = wpre_ref[2 * F:3 * F, :]
    x = x_ref[...]
    a_ref[...] = jnp.dot(x, wp1, preferred_element_type=jnp.float32)
    b_ref[...] = jnp.dot(x, wp2, preferred_element_type=jnp.float32)
    wec_ref[...] = jnp.dot(wedge_ref[...], wp3, preferred_element_type=jnp.float32)
    c0_ref[...] = (jnp.dot(bedge_ref[...], wp3, preferred_element_type=jnp.float32)
                   + bpre_ref[...])
    wlin = wlin_ref[...]
    p_ref[...] = jnp.dot(wpost_ref[...], wlin, preferred_element_type=jnp.float32)
    bout_ref[...] = (jnp.dot(bpost_ref[...], wlin, preferred_element_type=jnp.float32)
                     + blin_ref[...])


def _cmat_body(ea_ref, wec_ref, c0_ref, c_ref):
    c_ref[...] = (jnp.dot(ea_ref[...], wec_ref[...],
                          preferred_element_type=jnp.float32) + c0_ref[...])


def _sc_body(src_hbm, dst_hbm, b_hbm, c_hbm,
             cnt_hbm, s_hbm, s2_hbm, mx_hbm, mn_hbm,
             dbuf0, dbuf1, sbuf0, sbuf1, ldst, lsrc, lid,
             brow0, brow1, crow0, crow1,
             acc_s, acc_s2, acc_mx, acc_mn, acc_c, cv_ref, ex_ref,
             sem_b0, sem_c0, sem_b1, sem_c1):
    cid = lax.axis_index("c")
    sid = lax.axis_index("s")
    wid = sid * 2 + cid

    zf = jnp.zeros((16,), jnp.float32)
    zi = jnp.zeros((16,), jnp.int32)
    neg = jnp.full((16,), -3.0e38, jnp.float32)
    big = jnp.full((16,), 3.0e38, jnp.float32)
    iota16 = lax.iota(jnp.int32, 16)
    onesf = jnp.ones((16,), jnp.float32)
    lane0 = iota16 == 0

    def init_lists(i, _):
        o = i * 16
        ldst[pl.ds(o, 16)] = zi
        lsrc[pl.ds(o, 16)] = zi
        lid[pl.ds(o, 16)] = zi
        return 0
    lax.fori_loop(0, LLEN // 16, init_lists, 0)

    def round_body(r, _):
        kb = wid + 32 * r
        lo = kb * NPB
        lo128 = lo * F

        def init_acc(i, _):
            o = i * 16
            acc_s[pl.ds(o, 16)] = zf
            acc_s2[pl.ds(o, 16)] = zf
            return 0
        lax.fori_loop(0, (NPB + 1) * F // 16, init_acc, 0)

        def init_mxmn(i, _):
            o = i * 16
            for f in range(F // 16):
                acc_mx[f][pl.ds(o, 16)] = neg
                acc_mn[f][pl.ds(o, 16)] = big
            return 0
        lax.fori_loop(0, NPB + 1, init_mxmn, 0)

        def init_cnt(i, _):
            acc_c[pl.ds(i * 16, 16)] = zf
            return 0
        lax.fori_loop(0, (NPB + 16) // 16, init_cnt, 0)

        def scan_inner(db, sb, base, pv0):
            def scan_v(j, pv):
                d = db[pl.ds(j * 16, 16)]
                s = sb[pl.ds(j * 16, 16)]
                m = (d >= lo) & (d < lo + NPB)
                csum = plsc.cumsum(m.astype(jnp.int32))
                posn = pv + csum - 1
                m2 = m & (posn < LCAP)
                plsc.store_scatter(ldst, [posn], d, mask=m2)
                plsc.store_scatter(lsrc, [posn], s, mask=m2)
                ids = base + j * 16 + iota16
                plsc.store_scatter(lid, [posn], ids, mask=m2)
                return pv + plsc.all_reduce_population_count(m)
            return lax.fori_loop(0, CH // 16, scan_v, pv0)

        def issue_scan(ch, db, sb, semd, sems):
            base = jnp.minimum(ch, NCH - 1) * CH
            pltpu.async_copy(dst_hbm.at[pl.ds(base, CH)], db, semd)
            pltpu.async_copy(src_hbm.at[pl.ds(base, CH)], sb, sems)

        def wait_scan(db, sb, semd, sems):
            pltpu.make_async_copy(dst_hbm.at[pl.ds(0, CH)], db, semd).wait()
            pltpu.make_async_copy(src_hbm.at[pl.ds(0, CH)], sb, sems).wait()

        issue_scan(0, dbuf0, sbuf0, sem_b0, sem_c0)

        def scan_pair(i, pv):
            c0 = 2 * i
            issue_scan(c0 + 1, dbuf1, sbuf1, sem_b1, sem_c1)
            wait_scan(dbuf0, sbuf0, sem_b0, sem_c0)
            pv = scan_inner(dbuf0, sbuf0, c0 * CH, pv)
            issue_scan(c0 + 2, dbuf0, sbuf0, sem_b0, sem_c0)
            wait_scan(dbuf1, sbuf1, sem_b1, sem_c1)
            pv = scan_inner(dbuf1, sbuf1, (c0 + 1) * CH, pv)
            return pv

        ptrv = lax.fori_loop(0, NCH // 2, scan_pair, zi)
        wait_scan(dbuf0, sbuf0, sem_b0, sem_c0)
        n = jnp.minimum(ptrv[0], LCAP)

        junkv = jnp.full((16,), lo + NPB, jnp.int32)

        def padk(k, _):
            posn = n + k * 16 + iota16
            plsc.store_scatter(ldst, [posn], junkv, mask=posn < LLEN)
            return 0
        lax.fori_loop(0, 2 * G // 16, padk, 0)

        nch = (n + G - 1) // G
        nhalf = (nch + 1) // 2

        def issue_gather(g, br, cr, semb, semc):
            gb = g * G
            pltpu.async_copy(b_hbm.at[lsrc.at[pl.ds(gb, G)]], br, semb)
            pltpu.async_copy(c_hbm.at[lid.at[pl.ds(gb, G)]], cr, semc)

        def wait_gather(br, cr, semb, semc):
            pltpu.make_async_copy(b_hbm.at[lsrc.at[pl.ds(0, G)]], br, semb).wait()
            pltpu.make_async_copy(c_hbm.at[lid.at[pl.ds(0, G)]], cr, semc).wait()

        def process(br, cr, gb):
            cfs = [jnp.full((16,), f * 16, jnp.int32) + iota16
                   for f in range(F // 16)]

            def per_group(gi, _):
                gbase = gb + gi * 16
                for es in range(16):
                    ev = jnp.full((16,), gbase + es, jnp.int32)
                    rel = plsc.load_gather(ldst, [ev]) - lo
                    plsc.addupdate_scatter(acc_c, [rel], onesf, mask=lane0)
                    rel128 = rel * F
                    addr16 = rel * 16 + iota16
                    erow = gi * 16 + es
                    nf = F // 16
                    ts = [br[erow, pl.ds(f * 16, 16)]
                          + cr[erow, pl.ds(f * 16, 16)] for f in range(nf)]
                    for f in range(nf):
                        plsc.addupdate_scatter(acc_s, [rel128 + cfs[f]], ts[f])
                    for f in range(nf):
                        plsc.addupdate_scatter(acc_s2, [rel128 + cfs[f]],
                                               ts[f] * ts[f])
                    mxs = [plsc.load_gather(acc_mx[f], [addr16])
                           for f in range(nf)]
                    for f in range(nf):
                        plsc.store_scatter(acc_mx[f], [addr16],
                                           jnp.maximum(mxs[f], ts[f]))
                    mns = [plsc.load_gather(acc_mn[f], [addr16])
                           for f in range(nf)]
                    for f in range(nf):
                        plsc.store_scatter(acc_mn[f], [addr16],
                                           jnp.minimum(mns[f], ts[f]))
                return 0
            lax.fori_loop(0, G // 16, per_group, 0)

        issue_gather(0, brow0, crow0, sem_b0, sem_c0)

        def acc_pair(i, _):
            g0 = 2 * i
            issue_gather(g0 + 1, brow1, crow1, sem_b1, sem_c1)
            wait_gather(brow0, crow0, sem_b0, sem_c0)
            process(brow0, crow0, g0 * G)
            issue_gather(g0 + 2, brow0, crow0, sem_b0, sem_c0)
            wait_gather(brow1, crow1, sem_b1, sem_c1)
            process(brow1, crow1, (g0 + 1) * G)
            return 0

        lax.fori_loop(0, nhalf, acc_pair, 0)
        wait_gather(brow0, crow0, sem_b0, sem_c0)

        pltpu.sync_copy(acc_s.at[pl.ds(0, NPB * F)], s_hbm.at[pl.ds(lo128, NPB * F)])
        pltpu.sync_copy(acc_s2.at[pl.ds(0, NPB * F)], s2_hbm.at[pl.ds(lo128, NPB * F)])
        for f in range(F // 16):
            pltpu.sync_copy(acc_mx[f].at[pl.ds(0, NPB * 16)],
                            mx_hbm.at[pl.ds(f * NPAD * 16 + lo * 16, NPB * 16)])
            pltpu.sync_copy(acc_mn[f].at[pl.ds(0, NPB * 16)],
                            mn_hbm.at[pl.ds(f * NPAD * 16 + lo * 16, NPB * 16)])
        pltpu.sync_copy(acc_c.at[pl.ds(0, NPB)], cnt_hbm.at[pl.ds(lo, NPB)])
        return 0

    lax.fori_loop(0, NROUND, round_body, 0)


def _post_body(x_ref, a_ref, cnt_ref, s_ref, s2_ref, mx_ref, mn_ref,
               p_ref, bout_ref, o_ref):
    cnt = cnt_ref[...]
    has = cnt > 0.0
    cntc = jnp.maximum(cnt, 1.0)
    inv = 1.0 / cntc
    a = a_ref[...]
    s = s_ref[...]
    mt = s * inv
    mean = jnp.where(has, a + mt, 0.0)
    mx = jnp.where(has, a + mx_ref[...], 0.0)
    mn = jnp.where(has, a + mn_ref[...], 0.0)
    var = s2_ref[...] * inv - mt * mt
    std = jnp.sqrt(jnp.maximum(var, 0.0) + 1e-5)
    agg = jnp.concatenate([mean, mx, mn, std], axis=1)
    degl = jnp.log(cntc + 1.0)
    s_amp = degl * (1.0 / AVG_DEG_LOG)
    s_att = AVG_DEG_LOG / degl
    p = p_ref[...]
    out = jnp.dot(x_ref[...], p[0:F, :], preferred_element_type=jnp.float32)
    out += jnp.dot(agg, p[F:5 * F, :], preferred_element_type=jnp.float32)
    out += s_amp * jnp.dot(agg, p[5 * F:9 * F, :], preferred_element_type=jnp.float32)
    out += s_att * jnp.dot(agg, p[9 * F:13 * F, :], preferred_element_type=jnp.float32)
    o_ref[...] = out + bout_ref[...]


def kernel(x, edge_index, edge_attr, W_edge, b_edge, W_pre, b_pre,
           W_post, b_post, W_lin, b_lin):
    x_pad = jnp.pad(x, ((0, NPAD - N_NODES), (0, 0)))

    prep = pl.pallas_call(
        _prep_body,
        out_shape=[
            jax.ShapeDtypeStruct((NPAD, F), jnp.float32),
            jax.ShapeDtypeStruct((NPAD, F), jnp.float32),
            jax.ShapeDtypeStruct((10, F), jnp.float32),
            jax.ShapeDtypeStruct((1, F), jnp.float32),
            jax.ShapeDtypeStruct((13 * F, F), jnp.float32),
            jax.ShapeDtypeStruct((1, F), jnp.float32),
        ],
    )
    a_mat, b_mat, w_ec, c0, p_mat, b_out = prep(
        x_pad, W_pre, W_edge, b_edge.reshape(1, F), b_pre.reshape(1, F),
        W_post, b_post.reshape(1, F), W_lin, b_lin.reshape(1, F))

    cmat = pl.pallas_call(
        _cmat_body,
        grid=(N_EDGES // EBLK,),
        in_specs=[
            pl.BlockSpec((EBLK, 10), lambda i: (i, 0)),
            pl.BlockSpec((10, F), lambda i: (0, 0)),
            pl.BlockSpec((1, F), lambda i: (0, 0)),
        ],
        out_specs=pl.BlockSpec((EBLK, F), lambda i: (i, 0)),
        out_shape=jax.ShapeDtypeStruct((N_EDGES, F), jnp.float32),
    )
    c_mat = cmat(edge_attr, w_ec, c0)

    mesh = plsc.VectorSubcoreMesh(core_axis_name="c", subcore_axis_name="s")
    sc = pl.kernel(
        _sc_body,
        out_type=[
            jax.ShapeDtypeStruct((NPAD,), jnp.float32),
            jax.ShapeDtypeStruct((NPAD * F,), jnp.float32),
            jax.ShapeDtypeStruct((NPAD * F,), jnp.float32),
            jax.ShapeDtypeStruct((NPAD * F,), jnp.float32),
            jax.ShapeDtypeStruct((NPAD * F,), jnp.float32),
        ],
        mesh=mesh,
        scratch_types=[
            pltpu.VMEM((CH,), jnp.int32),
            pltpu.VMEM((CH,), jnp.int32),
            pltpu.VMEM((CH,), jnp.int32),
            pltpu.VMEM((CH,), jnp.int32),
            pltpu.VMEM((LLEN,), jnp.int32),
            pltpu.VMEM((LLEN,), jnp.int32),
            pltpu.VMEM((LLEN,), jnp.int32),
            pltpu.VMEM((G, F), jnp.float32),
            pltpu.VMEM((G, F), jnp.float32),
            pltpu.VMEM((G, F), jnp.float32),
            pltpu.VMEM((G, F), jnp.float32),
            pltpu.VMEM(((NPB + 1) * F,), jnp.float32),
            pltpu.VMEM(((NPB + 1) * F,), jnp.float32),
            [pltpu.VMEM(((NPB + 1) * 16,), jnp.float32)
             for _ in range(F // 16)],
            [pltpu.VMEM(((NPB + 1) * 16,), jnp.float32)
             for _ in range(F // 16)],
            pltpu.VMEM((NPB + 16,), jnp.float32),
            pltpu.VMEM((16,), jnp.int32),
            pltpu.VMEM((16,), jnp.int32),
            pltpu.SemaphoreType.DMA,
            pltpu.SemaphoreType.DMA,
            pltpu.SemaphoreType.DMA,
            pltpu.SemaphoreType.DMA,
        ],
        compiler_params=pltpu.CompilerParams(needs_layout_passes=False),
    )
    cnt, s_flat, s2_flat, mx_flat, mn_flat = sc(
        edge_index[0], edge_index[1], b_mat, c_mat)

    post = pl.pallas_call(
        _post_body,
        grid=(NPAD // NBLK4,),
        in_specs=[
            pl.BlockSpec((NBLK4, F), lambda i: (i, 0)),
            pl.BlockSpec((NBLK4, F), lambda i: (i, 0)),
            pl.BlockSpec((NBLK4, 1), lambda i: (i, 0)),
            pl.BlockSpec((NBLK4, F), lambda i: (i, 0)),
            pl.BlockSpec((NBLK4, F), lambda i: (i, 0)),
            pl.BlockSpec((NBLK4, F), lambda i: (i, 0)),
            pl.BlockSpec((NBLK4, F), lambda i: (i, 0)),
            pl.BlockSpec((13 * F, F), lambda i: (0, 0)),
            pl.BlockSpec((1, F), lambda i: (0, 0)),
        ],
        out_specs=pl.BlockSpec((NBLK4, F), lambda i: (i, 0)),
        out_shape=jax.ShapeDtypeStruct((NPAD, F), jnp.float32),
    )
    mx2d = mx_flat.reshape(F // 16, NPAD, 16).transpose(1, 0, 2).reshape(NPAD, F)
    mn2d = mn_flat.reshape(F // 16, NPAD, 16).transpose(1, 0, 2).reshape(NPAD, F)
    out = post(x_pad, a_mat, cnt.reshape(NPAD, 1),
               s_flat.reshape(NPAD, F), s2_flat.reshape(NPAD, F),
               mx2d, mn2d, p_mat, b_out)
    return out[:N_NODES]


# speedup vs baseline: 1.2822x; 1.2493x over previous
import functools

import jax
import jax.numpy as jnp
from jax import lax
from jax.experimental import pallas as pl
from jax.experimental.pallas import tpu as pltpu
from jax.experimental.pallas import tpu_sc as plsc
import numpy as np

F = 128
N_NODES = 10000
N_EDGES = 320000
AVG_DEG_LOG = float(np.log(33.0))

NB = 96
NPB = 112
NPAD = NB * NPB
NROUND = NB // 32
LCAP = 6144
G = 64
LLEN = LCAP + 3 * G + 16
CH = 3200
NCH = N_EDGES // CH
EBLK = 4000
NBLK4 = 768


def _prep_body(x_ref, wpre_ref, wedge_ref, bedge_ref, bpre_ref, wpost_ref,
               bpost_ref, wlin_ref, blin_ref,
               a_ref, b_ref, wec_ref, c0_ref, p_ref, bout_ref):
    wp1 = wpre_ref[0:F, :]
    wp2 = wpre_ref[F:2 * F, :]
    wp3 = wpre_ref[2 * F:3 * F, :]
    x = x_ref[...]
    a_ref[...] = jnp.dot(x, wp1, preferred_element_type=jnp.float32)
    b_ref[...] = jnp.dot(x, wp2, preferred_element_type=jnp.float32)
    wec_ref[...] = jnp.dot(wedge_ref[...], wp3, preferred_element_type=jnp.float32)
    c0_ref[...] = (jnp.dot(bedge_ref[...], wp3, preferred_element_type=jnp.float32)
                   + bpre_ref[...])
    wlin = wlin_ref[...]
    p_ref[...] = jnp.dot(wpost_ref[...], wlin, preferred_element_type=jnp.float32)
    bout_ref[...] = (jnp.dot(bpost_ref[...], wlin, preferred_element_type=jnp.float32)
                     + blin_ref[...])


def _cmat_body(ea_ref, wec_ref, c0_ref, c_ref):
    c_ref[...] = (jnp.dot(ea_ref[...], wec_ref[...],
                          preferred_element_type=jnp.float32) + c0_ref[...])


def _sc_body(src_hbm, dst_hbm, b_hbm, c_hbm,
             cnt_hbm, s_hbm, s2_hbm, mx_hbm, mn_hbm,
             dbuf0, dbuf1, sbuf0, sbuf1, ldst, lsrc, lid,
             brow0, brow1, crow0, crow1,
             acc_s, acc_s2, acc_mx, acc_mn, acc_c, cv_ref, ex_ref,
             sem_b0, sem_c0, sem_b1, sem_c1):
    cid = lax.axis_index("c")
    sid = lax.axis_index("s")
    wid = sid * 2 + cid

    zf = jnp.zeros((16,), jnp.float32)
    zi = jnp.zeros((16,), jnp.int32)
    neg = jnp.full((16,), -3.0e38, jnp.float32)
    big = jnp.full((16,), 3.0e38, jnp.float32)
    iota16 = lax.iota(jnp.int32, 16)
    onesf = jnp.ones((16,), jnp.float32)
    lane0 = iota16 == 0

    def init_lists(i, _):
        o = i * 16
        ldst[pl.ds(o, 16)] = zi
        lsrc[pl.ds(o, 16)] = zi
        lid[pl.ds(o, 16)] = zi
        return 0
    lax.fori_loop(0, LLEN // 16, init_lists, 0)

    def round_body(r, _):
        kb = wid + 32 * r
        lo = kb * NPB
        lo128 = lo * F

        def init_acc(i, _):
            o = i * 16
            acc_s[pl.ds(o, 16)] = zf
            acc_s2[pl.ds(o, 16)] = zf
            return 0
        lax.fori_loop(0, (NPB + 1) * F // 16, init_acc, 0)

        def init_mxmn(i, _):
            o = i * 16
            for f in range(F // 16):
                acc_mx[f][pl.ds(o, 16)] = neg
                acc_mn[f][pl.ds(o, 16)] = big
            return 0
        lax.fori_loop(0, NPB + 1, init_mxmn, 0)

        def init_cnt(i, _):
            acc_c[pl.ds(i * 16, 16)] = zf
            return 0
        lax.fori_loop(0, (NPB + 16) // 16, init_cnt, 0)

        def scan_inner(db, sb, base, pv0):
            def scan_v(j, pv):
                o = j * 32
                d0 = db[pl.ds(o, 16)]
                s0 = sb[pl.ds(o, 16)]
                d1 = db[pl.ds(o + 16, 16)]
                s1 = sb[pl.ds(o + 16, 16)]
                m0 = (d0 >= lo) & (d0 < lo + NPB)
                m1 = (d1 >= lo) & (d1 < lo + NPB)
                c0 = plsc.cumsum(m0.astype(jnp.int32))
                c1 = plsc.cumsum(m1.astype(jnp.int32))
                p0 = pv + plsc.all_reduce_population_count(m0)
                posn0 = pv + c0 - 1
                posn1 = p0 + c1 - 1
                mm0 = m0 & (posn0 < LCAP)
                mm1 = m1 & (posn1 < LCAP)
                plsc.store_scatter(ldst, [posn0], d0, mask=mm0)
                plsc.store_scatter(lsrc, [posn0], s0, mask=mm0)
                plsc.store_scatter(lid, [posn0], base + o + iota16, mask=mm0)
                plsc.store_scatter(ldst, [posn1], d1, mask=mm1)
                plsc.store_scatter(lsrc, [posn1], s1, mask=mm1)
                plsc.store_scatter(lid, [posn1], base + o + 16 + iota16,
                                   mask=mm1)
                return p0 + plsc.all_reduce_population_count(m1)
            return lax.fori_loop(0, CH // 32, scan_v, pv0)

        def issue_scan(ch, db, sb, semd, sems):
            base = jnp.minimum(ch, NCH - 1) * CH
            pltpu.async_copy(dst_hbm.at[pl.ds(base, CH)], db, semd)
            pltpu.async_copy(src_hbm.at[pl.ds(base, CH)], sb, sems)

        def wait_scan(db, sb, semd, sems):
            pltpu.make_async_copy(dst_hbm.at[pl.ds(0, CH)], db, semd).wait()
            pltpu.make_async_copy(src_hbm.at[pl.ds(0, CH)], sb, sems).wait()

        issue_scan(0, dbuf0, sbuf0, sem_b0, sem_c0)

        def scan_pair(i, pv):
            c0 = 2 * i
            issue_scan(c0 + 1, dbuf1, sbuf1, sem_b1, sem_c1)
            wait_scan(dbuf0, sbuf0, sem_b0, sem_c0)
            pv = scan_inner(dbuf0, sbuf0, c0 * CH, pv)
            issue_scan(c0 + 2, dbuf0, sbuf0, sem_b0, sem_c0)
            wait_scan(dbuf1, sbuf1, sem_b1, sem_c1)
            pv = scan_inner(dbuf1, sbuf1, (c0 + 1) * CH, pv)
            return pv

        ptrv = lax.fori_loop(0, NCH // 2, scan_pair, zi)
        wait_scan(dbuf0, sbuf0, sem_b0, sem_c0)
        n = jnp.minimum(ptrv[0], LCAP)

        junkv = jnp.full((16,), lo + NPB, jnp.int32)

        def padk(k, _):
            posn = n + k * 16 + iota16
            plsc.store_scatter(ldst, [posn], junkv, mask=posn < LLEN)
            return 0
        lax.fori_loop(0, 2 * G // 16, padk, 0)

        nch = (n + G - 1) // G
        nhalf = (nch + 1) // 2

        def issue_gather(g, br, cr, semb, semc):
            gb = g * G
            pltpu.async_copy(b_hbm.at[lsrc.at[pl.ds(gb, G)]], br, semb)
            pltpu.async_copy(c_hbm.at[lid.at[pl.ds(gb, G)]], cr, semc)

        def wait_gather(br, cr, semb, semc):
            pltpu.make_async_copy(b_hbm.at[lsrc.at[pl.ds(0, G)]], br, semb).wait()
            pltpu.make_async_copy(c_hbm.at[lid.at[pl.ds(0, G)]], cr, semc).wait()

        def process(br, cr, gb):
            cfs = [jnp.full((16,), f * 16, jnp.int32) + iota16
                   for f in range(F // 16)]

            def per_group(gi, _):
                gbase = gb + gi * 16
                for es in range(16):
                    ev = jnp.full((16,), gbase + es, jnp.int32)
                    rel = plsc.load_gather(ldst, [ev]) - lo
                    plsc.addupdate_scatter(acc_c, [rel], onesf, mask=lane0)
                    rel128 = rel * F
                    addr16 = rel * 16 + iota16
                    erow = gi * 16 + es
                    nf = F // 16
                    ts = [br[erow, pl.ds(f * 16, 16)]
                          + cr[erow, pl.ds(f * 16, 16)] for f in range(nf)]
                    for f in range(nf):
                        plsc.addupdate_scatter(acc_s, [rel128 + cfs[f]], ts[f])
                    for f in range(nf):
                        plsc.addupdate_scatter(acc_s2, [rel128 + cfs[f]],
                                               ts[f] * ts[f])
                    mxs = [plsc.load_gather(acc_mx[f], [addr16])
                           for f in range(nf)]
                    for f in range(nf):
                        plsc.store_scatter(acc_mx[f], [addr16],
                                           jnp.maximum(mxs[f], ts[f]))
                    mns = [plsc.load_gather(acc_mn[f], [addr16])
                           for f in range(nf)]
                    for f in range(nf):
                        plsc.store_scatter(acc_mn[f], [addr16],
                                           jnp.minimum(mns[f], ts[f]))
                return 0
            lax.fori_loop(0, G // 16, per_group, 0)

        issue_gather(0, brow0, crow0, sem_b0, sem_c0)

        def acc_pair(i, _):
            g0 = 2 * i
            issue_gather(g0 + 1, brow1, crow1, sem_b1, sem_c1)
            wait_gather(brow0, crow0, sem_b0, sem_c0)
            process(brow0, crow0, g0 * G)
            issue_gather(g0 + 2, brow0, crow0, sem_b0, sem_c0)
            wait_gather(brow1, crow1, sem_b1, sem_c1)
            process(brow1, crow1, (g0 + 1) * G)
            return 0

        lax.fori_loop(0, nhalf, acc_pair, 0)
        wait_gather(brow0, crow0, sem_b0, sem_c0)

        pltpu.sync_copy(acc_s.at[pl.ds(0, NPB * F)], s_hbm.at[pl.ds(lo128, NPB * F)])
        pltpu.sync_copy(acc_s2.at[pl.ds(0, NPB * F)], s2_hbm.at[pl.ds(lo128, NPB * F)])
        for f in range(F // 16):
            pltpu.sync_copy(acc_mx[f].at[pl.ds(0, NPB * 16)],
                            mx_hbm.at[pl.ds(f * NPAD * 16 + lo * 16, NPB * 16)])
            pltpu.sync_copy(acc_mn[f].at[pl.ds(0, NPB * 16)],
                            mn_hbm.at[pl.ds(f * NPAD * 16 + lo * 16, NPB * 16)])
        pltpu.sync_copy(acc_c.at[pl.ds(0, NPB)], cnt_hbm.at[pl.ds(lo, NPB)])
        return 0

    lax.fori_loop(0, NROUND, round_body, 0)


def _post_body(x_ref, a_ref, cnt_ref, s_ref, s2_ref, mx_ref, mn_ref,
               p_ref, bout_ref, o_ref):
    cnt = cnt_ref[...]
    has = cnt > 0.0
    cntc = jnp.maximum(cnt, 1.0)
    inv = 1.0 / cntc
    a = a_ref[...]
    s = s_ref[...]
    mt = s * inv
    mean = jnp.where(has, a + mt, 0.0)
    mx = jnp.where(has, a + mx_ref[...], 0.0)
    mn = jnp.where(has, a + mn_ref[...], 0.0)
    var = s2_ref[...] * inv - mt * mt
    std = jnp.sqrt(jnp.maximum(var, 0.0) + 1e-5)
    agg = jnp.concatenate([mean, mx, mn, std], axis=1)
    degl = jnp.log(cntc + 1.0)
    s_amp = degl * (1.0 / AVG_DEG_LOG)
    s_att = AVG_DEG_LOG / degl
    p = p_ref[...]
    out = jnp.dot(x_ref[...], p[0:F, :], preferred_element_type=jnp.float32)
    out += jnp.dot(agg, p[F:5 * F, :], preferred_element_type=jnp.float32)
    out += s_amp * jnp.dot(agg, p[5 * F:9 * F, :], preferred_element_type=jnp.float32)
    out += s_att * jnp.dot(agg, p[9 * F:13 * F, :], preferred_element_type=jnp.float32)
    o_ref[...] = out + bout_ref[...]


def kernel(x, edge_index, edge_attr, W_edge, b_edge, W_pre, b_pre,
           W_post, b_post, W_lin, b_lin):
    x_pad = jnp.pad(x, ((0, NPAD - N_NODES), (0, 0)))

    prep = pl.pallas_call(
        _prep_body,
        out_shape=[
            jax.ShapeDtypeStruct((NPAD, F), jnp.float32),
            jax.ShapeDtypeStruct((NPAD, F), jnp.float32),
            jax.ShapeDtypeStruct((10, F), jnp.float32),
            jax.ShapeDtypeStruct((1, F), jnp.float32),
            jax.ShapeDtypeStruct((13 * F, F), jnp.float32),
            jax.ShapeDtypeStruct((1, F), jnp.float32),
        ],
    )
    a_mat, b_mat, w_ec, c0, p_mat, b_out = prep(
        x_pad, W_pre, W_edge, b_edge.reshape(1, F), b_pre.reshape(1, F),
        W_post, b_post.reshape(1, F), W_lin, b_lin.reshape(1, F))

    cmat = pl.pallas_call(
        _cmat_body,
        grid=(N_EDGES // EBLK,),
        in_specs=[
            pl.BlockSpec((EBLK, 10), lambda i: (i, 0)),
            pl.BlockSpec((10, F), lambda i: (0, 0)),
            pl.BlockSpec((1, F), lambda i: (0, 0)),
        ],
        out_specs=pl.BlockSpec((EBLK, F), lambda i: (i, 0)),
        out_shape=jax.ShapeDtypeStruct((N_EDGES, F), jnp.float32),
    )
    c_mat = cmat(edge_attr, w_ec, c0)

    mesh = plsc.VectorSubcoreMesh(core_axis_name="c", subcore_axis_name="s")
    sc = pl.kernel(
        _sc_body,
        out_type=[
            jax.ShapeDtypeStruct((NPAD,), jnp.float32),
            jax.ShapeDtypeStruct((NPAD * F,), jnp.float32),
            jax.ShapeDtypeStruct((NPAD * F,), jnp.float32),
            jax.ShapeDtypeStruct((NPAD * F,), jnp.float32),
            jax.ShapeDtypeStruct((NPAD * F,), jnp.float32),
        ],
        mesh=mesh,
        scratch_types=[
            pltpu.VMEM((CH,), jnp.int32),
            pltpu.VMEM((CH,), jnp.int32),
            pltpu.VMEM((CH,), jnp.int32),
            pltpu.VMEM((CH,), jnp.int32),
            pltpu.VMEM((LLEN,), jnp.int32),
            pltpu.VMEM((LLEN,), jnp.int32),
            pltpu.VMEM((LLEN,), jnp.int32),
            pltpu.VMEM((G, F), jnp.float32),
            pltpu.VMEM((G, F), jnp.float32),
            pltpu.VMEM((G, F), jnp.float32),
            pltpu.VMEM((G, F), jnp.float32),
            pltpu.VMEM(((NPB + 1) * F,), jnp.float32),
            pltpu.VMEM(((NPB + 1) * F,), jnp.float32),
            [pltpu.VMEM(((NPB + 1) * 16,), jnp.float32)
             for _ in range(F // 16)],
            [pltpu.VMEM(((NPB + 1) * 16,), jnp.float32)
             for _ in range(F // 16)],
            pltpu.VMEM((NPB + 16,), jnp.float32),
            pltpu.VMEM((16,), jnp.int32),
            pltpu.VMEM((16,), jnp.int32),
            pltpu.SemaphoreType.DMA,
            pltpu.SemaphoreType.DMA,
            pltpu.SemaphoreType.DMA,
            pltpu.SemaphoreType.DMA,
        ],
        compiler_params=pltpu.CompilerParams(needs_layout_passes=False),
    )
    cnt, s_flat, s2_flat, mx_flat, mn_flat = sc(
        edge_index[0], edge_index[1], b_mat, c_mat)

    post = pl.pallas_call(
        _post_body,
        grid=(NPAD // NBLK4,),
        in_specs=[
            pl.BlockSpec((NBLK4, F), lambda i: (i, 0)),
            pl.BlockSpec((NBLK4, F), lambda i: (i, 0)),
            pl.BlockSpec((NBLK4, 1), lambda i: (i, 0)),
            pl.BlockSpec((NBLK4, F), lambda i: (i, 0)),
            pl.BlockSpec((NBLK4, F), lambda i: (i, 0)),
            pl.BlockSpec((NBLK4, F), lambda i: (i, 0)),
            pl.BlockSpec((NBLK4, F), lambda i: (i, 0)),
            pl.BlockSpec((13 * F, F), lambda i: (0, 0)),
            pl.BlockSpec((1, F), lambda i: (0, 0)),
        ],
        out_specs=pl.BlockSpec((NBLK4, F), lambda i: (i, 0)),
        out_shape=jax.ShapeDtypeStruct((NPAD, F), jnp.float32),
    )
    mx2d = mx_flat.reshape(F // 16, NPAD, 16).transpose(1, 0, 2).reshape(NPAD, F)
    mn2d = mn_flat.reshape(F // 16, NPAD, 16).transpose(1, 0, 2).reshape(NPAD, F)
    out = post(x_pad, a_mat, cnt.reshape(NPAD, 1),
               s_flat.reshape(NPAD, F), s2_flat.reshape(NPAD, F),
               mx2d, mn2d, p_mat, b_out)
    return out[:N_NODES]
